# Initial kernel scaffold; baseline (speedup 1.0000x reference)
import jax
import jax.numpy as jnp
from jax.experimental import pallas as pl


def kernel(output, confidence_threshold):
    raise NotImplementedError("write your pallas kernel here")


# speedup vs baseline: 1.3655x; 1.3655x over previous
import jax
import jax.numpy as jnp
from jax.experimental import pallas as pl
from jax.experimental.pallas import tpu as pltpu

_ANCHOR_H = 19.15
_ANCHOR_W = 85.72
_CELL = 32.0
_NA = 3
_THETA_MARGIN = 180.0 / _NA
_H = 160
_W = 160


def _decode_body(x_ref, dec_ref):
    x = x_ref[0]
    hw_i = jax.lax.broadcasted_iota(jnp.int32, (1, x.shape[1]), 1)
    iy_i = hw_i // _W
    ix = (hw_i - iy_i * _W).astype(jnp.float32)
    iy = iy_i.astype(jnp.float32)
    for a in range(_NA):
        b = a * 6
        dec_ref[0, b + 0:b + 1, :] = jax.nn.sigmoid(x[b + 0:b + 1])
        dec_ref[0, b + 1:b + 2, :] = (ix + jax.nn.sigmoid(x[b + 1:b + 2])) * _CELL
        dec_ref[0, b + 2:b + 3, :] = (iy + jax.nn.sigmoid(x[b + 2:b + 3])) * _CELL
        dec_ref[0, b + 3:b + 4, :] = _ANCHOR_W * jnp.exp(x[b + 3:b + 4])
        dec_ref[0, b + 4:b + 5, :] = _ANCHOR_H * jnp.exp(x[b + 4:b + 5])
        dec_ref[0, b + 5:b + 6, :] = (a + jax.nn.sigmoid(x[b + 5:b + 6])) * _THETA_MARGIN


def kernel(output, confidence_threshold):
    N, C, H, W = output.shape
    HW = H * W
    x = output.reshape(N, C, HW)
    dec = pl.pallas_call(
        _decode_body,
        grid=(N,),
        in_specs=[pl.BlockSpec((1, C, HW), lambda n: (n, 0, 0))],
        out_specs=pl.BlockSpec((1, C, HW), lambda n: (n, 0, 0)),
        out_shape=jax.ShapeDtypeStruct((N, C, HW), jnp.float32),
        compiler_params=pltpu.CompilerParams(
            dimension_semantics=("parallel",),
        ),
    )(x)
    boxes = dec.transpose(0, 2, 1).reshape(-1, 6)
    flat_mask = boxes[:, 0] >= confidence_threshold
    order = jnp.argsort(jnp.logical_not(flat_mask))
    keep = flat_mask[order].astype(boxes.dtype)[:, None]
    return boxes[order] * keep
